# Initial kernel scaffold; baseline (speedup 1.0000x reference)
import jax
import jax.numpy as jnp
from jax.experimental import pallas as pl


def kernel(fmap, W_base, b_base, W_cls, b_cls, W_reg, b_reg, img_h, img_w):
    raise NotImplementedError("write your pallas kernel here")


# speedup vs baseline: 24.0428x; 24.0428x over previous
import numpy as np
import jax
import jax.numpy as jnp
from jax.experimental import pallas as pl
from jax.experimental.pallas import tpu as pltpu

_F32 = jnp.float32
_NPIX = 4096
_TILE = 512
_NT = _NPIX // _TILE
_PRE = 1000
_POST = 300
_PAD = 128

_scales = np.array([128.0, 256.0, 512.0], np.float32)
_ratios = np.array([0.5, 1.0, 2.0], np.float32)
_hr = np.sqrt(_ratios)
_wr = (np.float32(1.0) / _hr).astype(np.float32)
_ws = (_wr[:, None] * _scales[None, :]).reshape(-1).astype(np.float32)
_hs = (_hr[:, None] * _scales[None, :]).reshape(-1).astype(np.float32)
_BX1 = (-_ws / 2).astype(np.float32)
_BY1 = (-_hs / 2).astype(np.float32)
_BX2 = (_ws / 2).astype(np.float32)
_BY2 = (_hs / 2).astype(np.float32)


def _rpn_kernel(f_ref, wb_ref, bb_ref, wc_ref, bc_ref, wr_ref, br_ref,
                abase_ref, scal_ref, out_ref,
                fpad, s_ref, x1_ref, y1_ref, x2_ref, y2_ref, ar_ref):
    w_lim = scal_ref[0]
    h_lim = scal_ref[1]
    stride_x = w_lim / 64.0
    stride_y = h_lim / 64.0

    bax1 = abase_ref[0]
    bay1 = abase_ref[1]
    bax2 = abase_ref[2]
    bay2 = abase_ref[3]

    fpad[:, 0:_PAD] = jnp.zeros((256, _PAD), _F32)
    fpad[:, _PAD + _NPIX:_PAD + _NPIX + _PAD] = jnp.zeros((256, _PAD), _F32)
    fpad[:, _PAD:_PAD + _NPIX] = f_ref[:]

    for t in range(_NT):
        base = t * _TILE
        col1 = jax.lax.broadcasted_iota(jnp.int32, (1, _TILE), 1) + base
        xc1 = jax.lax.rem(col1, 64)
        acc = jnp.zeros((256, _TILE), _F32)
        for kh in range(3):
            for kw in range(3):
                sh = (kh - 1) * 64 + (kw - 1)
                src = fpad[:, _PAD + base + sh:_PAD + base + sh + _TILE]
                if kw == 0:
                    src = jnp.where(xc1 >= 1, src, 0.0)
                elif kw == 2:
                    src = jnp.where(xc1 <= 62, src, 0.0)
                acc = acc + jnp.dot(wb_ref[kh, kw], src,
                                    preferred_element_type=_F32)
        xtile = jnp.maximum(acc + bb_ref[:], 0.0)

        logits = jnp.dot(wc_ref[:], xtile, preferred_element_type=_F32) + bc_ref[:]
        s_ref[:, base:base + _TILE] = jax.nn.sigmoid(logits)

        o0 = jnp.dot(wr_ref[0], xtile, preferred_element_type=_F32) + br_ref[0]
        o1 = jnp.dot(wr_ref[1], xtile, preferred_element_type=_F32) + br_ref[1]
        o2 = jnp.dot(wr_ref[2], xtile, preferred_element_type=_F32) + br_ref[2]
        o3 = jnp.dot(wr_ref[3], xtile, preferred_element_type=_F32) + br_ref[3]

        jj = jax.lax.broadcasted_iota(jnp.int32, (9, _TILE), 1) + base
        xg = jax.lax.rem(jj, 64).astype(_F32) * stride_x
        yg = jax.lax.div(jj, 64).astype(_F32) * stride_y
        ax1 = xg + bax1
        ay1 = yg + bay1
        ax2 = xg + bax2
        ay2 = yg + bay2
        aw = ax2 - ax1
        ah = ay2 - ay1
        cx = ax1 + aw * 0.5
        cy = ay1 + ah * 0.5
        px = cx + o0 * aw
        py = cy + o1 * ah
        pw = jnp.exp(o2) * aw
        ph = jnp.exp(o3) * ah
        x1 = jnp.clip(px - pw * 0.5, 0.0, w_lim)
        y1 = jnp.clip(py - ph * 0.5, 0.0, h_lim)
        x2 = jnp.clip(px + pw * 0.5, 0.0, w_lim)
        y2 = jnp.clip(py + ph * 0.5, 0.0, h_lim)
        x1_ref[:, base:base + _TILE] = x1
        y1_ref[:, base:base + _TILE] = y1
        x2_ref[:, base:base + _TILE] = x2
        y2_ref[:, base:base + _TILE] = y2
        ar_ref[:, base:base + _TILE] = (x2 - x1) * (y2 - y1)

    S = s_ref[:]
    bits = jax.lax.bitcast_convert_type(S, jnp.int32)
    e2d = (jax.lax.broadcasted_iota(jnp.int32, (9, _NPIX), 1) * 9
           + jax.lax.broadcasted_iota(jnp.int32, (9, _NPIX), 0))

    def _bs(i, c):
        lo, hi = c
        mid = lo + jax.lax.div(hi - lo, 2)
        cnt = jnp.sum((bits >= mid).astype(jnp.int32))
        ge = cnt >= _PRE
        return (jnp.where(ge, mid, lo), jnp.where(ge, hi, mid))

    lo, hi = jax.lax.fori_loop(
        0, 31, _bs, (jnp.int32(0), jnp.int32(0x7F800000)))
    thr_bits = lo
    cnt_gt = jnp.sum((bits > thr_bits).astype(jnp.int32))
    need = _PRE - cnt_gt
    tie = bits == thr_bits

    def _es(i, c):
        lo2, hi2 = c
        mid = lo2 + jax.lax.div(hi2 - lo2, 2)
        c2 = jnp.sum((tie & (e2d <= mid)).astype(jnp.int32))
        ge = c2 >= need
        return (jnp.where(ge, lo2, mid), jnp.where(ge, mid, hi2))

    lo2, hi2 = jax.lax.fori_loop(
        0, 17, _es, (jnp.int32(-1), jnp.int32(9 * _NPIX - 1)))
    sel = (bits > thr_bits) | (tie & (e2d <= hi2))

    X1 = x1_ref[:]
    Y1 = y1_ref[:]
    X2 = x2_ref[:]
    Y2 = y2_ref[:]
    AR = ar_ref[:]
    valid = ((X2 - X1) >= 1e-3) & ((Y2 - Y1) >= 1e-3)
    s0 = jnp.where(sel & valid, S, -1.0)

    out_ref[:] = jnp.zeros((_POST, 8), _F32)

    def _nms(i, c):
        s, r = c
        m = jnp.max(s)
        act = m > 0.0
        em = jnp.min(jnp.where(s == m, e2d, jnp.int32(1 << 30)))
        pick = (s == m) & (e2d == em)

        def pz(a):
            return jnp.sum(jnp.where(pick, a, 0.0))

        bx1 = pz(X1)
        by1 = pz(Y1)
        bx2 = pz(X2)
        by2 = pz(Y2)
        am = pz(AR)
        ix1 = jnp.maximum(bx1, X1)
        iy1 = jnp.maximum(by1, Y1)
        ix2 = jnp.minimum(bx2, X2)
        iy2 = jnp.minimum(by2, Y2)
        inter = jnp.maximum(ix2 - ix1, 0.0) * jnp.maximum(iy2 - iy1, 0.0)
        iou = inter / (am + AR - inter + 1e-9)
        s2 = jnp.where(jnp.logical_and(act, iou > 0.7), -1.0, s)
        wr_ok = jnp.logical_and(act, r < _POST)

        @pl.when(wr_ok)
        def _():
            row = jnp.concatenate(
                [bx1.reshape(1, 1), by1.reshape(1, 1), bx2.reshape(1, 1),
                 by2.reshape(1, 1), m.reshape(1, 1),
                 jnp.zeros((1, 3), _F32)], axis=1)
            out_ref[pl.dslice(r, 1), :] = row

        return (s2, r + wr_ok.astype(jnp.int32))

    jax.lax.fori_loop(0, _POST, _nms, (s0, jnp.int32(0)))


def kernel(fmap, W_base, b_base, W_cls, b_cls, W_reg, b_reg, img_h, img_w):
    f2d = fmap.reshape(256, _NPIX)
    wb = jnp.transpose(W_base, (2, 3, 0, 1))
    bb = b_base.reshape(256, 1)
    wc = W_cls[:, :, 0, 0]
    bc = b_cls.reshape(9, 1)
    wr = jnp.transpose(W_reg[:, :, 0, 0].reshape(9, 4, 256), (1, 0, 2))
    br = jnp.transpose(b_reg.reshape(9, 4), (1, 0)).reshape(4, 9, 1)
    scal = jnp.stack([jnp.asarray(img_w, _F32), jnp.asarray(img_h, _F32)])
    abase = jnp.asarray(np.stack([_BX1, _BY1, _BX2, _BY2]).reshape(4, 9, 1))

    out = pl.pallas_call(
        _rpn_kernel,
        out_shape=jax.ShapeDtypeStruct((_POST, 8), _F32),
        in_specs=[
            pl.BlockSpec(memory_space=pltpu.VMEM),
            pl.BlockSpec(memory_space=pltpu.VMEM),
            pl.BlockSpec(memory_space=pltpu.VMEM),
            pl.BlockSpec(memory_space=pltpu.VMEM),
            pl.BlockSpec(memory_space=pltpu.VMEM),
            pl.BlockSpec(memory_space=pltpu.VMEM),
            pl.BlockSpec(memory_space=pltpu.VMEM),
            pl.BlockSpec(memory_space=pltpu.VMEM),
            pl.BlockSpec(memory_space=pltpu.SMEM),
        ],
        out_specs=pl.BlockSpec(memory_space=pltpu.VMEM),
        scratch_shapes=[
            pltpu.VMEM((256, _PAD + _NPIX + _PAD), _F32),
            pltpu.VMEM((9, _NPIX), _F32),
            pltpu.VMEM((9, _NPIX), _F32),
            pltpu.VMEM((9, _NPIX), _F32),
            pltpu.VMEM((9, _NPIX), _F32),
            pltpu.VMEM((9, _NPIX), _F32),
            pltpu.VMEM((9, _NPIX), _F32),
        ],
    )(f2d, wb, bb, wc, bc, wr, br, abase, scal)
    return out[:, :5]
